# Initial kernel scaffold; baseline (speedup 1.0000x reference)
import jax
import jax.numpy as jnp
from jax.experimental import pallas as pl


def kernel(src, lengths, table):
    raise NotImplementedError("write your pallas kernel here")


# speedup vs baseline: 2.5463x; 2.5463x over previous
import functools

import jax
import jax.numpy as jnp
from jax.experimental import pallas as pl
from jax.experimental.pallas import tpu as pltpu
from jax.experimental.pallas import tpu_sc as plsc

_NUM_LAYERS = 2
_S_LEN = 200
_B = 1024
_D = 64
_N_IDX = _S_LEN * _B
_W = 128


def _sc_gather(table, idx_flat):
    mesh = plsc.VectorSubcoreMesh(
        core_axis_name="core", subcore_axis_name="subcore"
    )

    @functools.partial(
        pl.kernel,
        out_type=jax.ShapeDtypeStruct((_N_IDX, _D), jnp.float32),
        mesh=mesh,
        compiler_params=pltpu.CompilerParams(use_tc_tiling_on_sc=False),
    )
    def k(table_hbm, idx_hbm, out_hbm):
        def body(i_vmem, o_vmem):
            pltpu.sync_copy(table_hbm.at[i_vmem.at[0]], o_vmem)

        pltpu.emit_pipeline(
            body,
            grid=(_N_IDX // _W,),
            in_specs=[pl.BlockSpec((1, _W), lambda i: (0, i))],
            out_specs=[pl.BlockSpec((_W, _D), lambda i: (i, 0))],
            core_axis_name=("core", "subcore"),
            dimension_semantics=(pltpu.PARALLEL,),
        )(idx_hbm, out_hbm)

    return k(table, idx_flat)


_BLK = 4096


def _tc_mean(emb_flat):
    def body(x_ref, o_ref):
        s = jnp.sum(x_ref[...], axis=0, keepdims=True) * (1.0 / _S_LEN)
        o_ref[...] = jnp.broadcast_to(s, (_NUM_LAYERS, _BLK))

    return pl.pallas_call(
        body,
        grid=(_B * _D // _BLK,),
        in_specs=[pl.BlockSpec((_S_LEN, _BLK), lambda i: (0, i))],
        out_specs=pl.BlockSpec((_NUM_LAYERS, _BLK), lambda i: (0, i)),
        out_shape=jax.ShapeDtypeStruct((_NUM_LAYERS, _B * _D), jnp.float32),
    )(emb_flat)


def kernel(src, lengths, table):
    del lengths
    idx = src[..., 0].astype(jnp.int32).reshape(1, _N_IDX)
    flat = _sc_gather(table, idx)
    memory_bank = flat.reshape(_S_LEN, _B, _D)
    enc_final = _tc_mean(flat.reshape(_S_LEN, _B * _D)).reshape(
        _NUM_LAYERS, _B, _D
    )
    return (enc_final, enc_final, memory_bank)


# speedup vs baseline: 2.7154x; 1.0664x over previous
import functools

import jax
import jax.numpy as jnp
from jax.experimental import pallas as pl
from jax.experimental.pallas import tpu as pltpu
from jax.experimental.pallas import tpu_sc as plsc

_NUM_LAYERS = 2
_S_LEN = 200
_B = 1024
_D = 64
_N_IDX = _S_LEN * _B
_W = 128


def _sc_gather(table, idx_flat):
    mesh = plsc.VectorSubcoreMesh(
        core_axis_name="core", subcore_axis_name="subcore"
    )

    @functools.partial(
        pl.kernel,
        out_type=jax.ShapeDtypeStruct((_N_IDX, _D), jnp.float32),
        mesh=mesh,
        compiler_params=pltpu.CompilerParams(use_tc_tiling_on_sc=False),
    )
    def k(table_hbm, idx_hbm, out_hbm):
        def body(i_vmem, o_vmem):
            pltpu.sync_copy(table_hbm.at[i_vmem.at[0]], o_vmem)

        pltpu.emit_pipeline(
            body,
            grid=(_N_IDX // _W,),
            in_specs=[pl.BlockSpec((1, _W), lambda i: (0, i))],
            out_specs=[pl.BlockSpec((_W, _D), lambda i: (i, 0))],
            core_axis_name=("core", "subcore"),
            dimension_semantics=(pltpu.PARALLEL,),
        )(idx_hbm, out_hbm)

    return k(table, idx_flat)


_BBLK = 256


def _tc_mean(emb):
    def body(x_ref, o_ref):
        s = jnp.sum(x_ref[...], axis=0, keepdims=True) * (1.0 / _S_LEN)
        o_ref[...] = jnp.broadcast_to(s, (_NUM_LAYERS, _BBLK, _D))

    return pl.pallas_call(
        body,
        grid=(_B // _BBLK,),
        in_specs=[pl.BlockSpec((_S_LEN, _BBLK, _D), lambda i: (0, i, 0))],
        out_specs=pl.BlockSpec((_NUM_LAYERS, _BBLK, _D), lambda i: (0, i, 0)),
        out_shape=jax.ShapeDtypeStruct((_NUM_LAYERS, _B, _D), jnp.float32),
    )(emb)


def kernel(src, lengths, table):
    del lengths
    idx = src[..., 0].astype(jnp.int32).reshape(1, _N_IDX)
    flat = _sc_gather(table, idx)
    memory_bank = flat.reshape(_S_LEN, _B, _D)
    enc_final = _tc_mean(memory_bank)
    return (enc_final, enc_final, memory_bank)


# speedup vs baseline: 2.8883x; 1.0637x over previous
import functools

import jax
import jax.numpy as jnp
from jax.experimental import pallas as pl
from jax.experimental.pallas import tpu as pltpu
from jax.experimental.pallas import tpu_sc as plsc

_NUM_LAYERS = 2
_S_LEN = 200
_B = 1024
_D = 64
_N_IDX = _S_LEN * _B
_W = 128


def _sc_gather(table, idx_flat):
    mesh = plsc.VectorSubcoreMesh(
        core_axis_name="core", subcore_axis_name="subcore"
    )

    @functools.partial(
        pl.kernel,
        out_type=jax.ShapeDtypeStruct((_N_IDX, _D), jnp.float32),
        mesh=mesh,
        compiler_params=pltpu.CompilerParams(use_tc_tiling_on_sc=False),
    )
    def k(table_hbm, idx_hbm, out_hbm):
        def body(i_vmem, o_vmem):
            pltpu.sync_copy(table_hbm.at[i_vmem.at[0]], o_vmem)

        pltpu.emit_pipeline(
            body,
            grid=(_N_IDX // _W,),
            in_specs=[pl.BlockSpec((1, _W), lambda i: (0, i))],
            out_specs=[pl.BlockSpec((_W, _D), lambda i: (i, 0))],
            core_axis_name=("core", "subcore"),
            dimension_semantics=(pltpu.PARALLEL,),
        )(idx_hbm, out_hbm)

    return k(table, idx_flat)


_QBLK = 64


def _tc_mean(emb2):
    def body(x_ref, o_ref):
        s = jnp.sum(x_ref[...], axis=0) * (1.0 / _S_LEN)
        s = s.reshape(2 * _QBLK, _D)
        o_ref[...] = jnp.broadcast_to(s[None], (_NUM_LAYERS, 2 * _QBLK, _D))

    return pl.pallas_call(
        body,
        grid=(_B // 2 // _QBLK,),
        in_specs=[pl.BlockSpec((_S_LEN, _QBLK, 2 * _D), lambda i: (0, i, 0))],
        out_specs=pl.BlockSpec(
            (_NUM_LAYERS, 2 * _QBLK, _D), lambda i: (0, i, 0)
        ),
        out_shape=jax.ShapeDtypeStruct((_NUM_LAYERS, _B, _D), jnp.float32),
    )(emb2)


def kernel(src, lengths, table):
    del lengths
    idx = src[..., 0].astype(jnp.int32).reshape(1, _N_IDX)
    flat = _sc_gather(table, idx)
    memory_bank = flat.reshape(_S_LEN, _B, _D)
    enc_final = _tc_mean(flat.reshape(_S_LEN, _B // 2, 2 * _D))
    return (enc_final, enc_final, memory_bank)
